# Initial kernel scaffold; baseline (speedup 1.0000x reference)
import jax
import jax.numpy as jnp
from jax.experimental import pallas as pl


def kernel(positions, goals, prev_positions, agent_momentum, collision_history, collisions):
    raise NotImplementedError("write your pallas kernel here")


# speedup vs baseline: 2.6524x; 2.6524x over previous
import dataclasses
import functools

import jax
import jax.numpy as jnp
from jax import lax
from jax.experimental import pallas as pl
from jax.experimental.pallas import tpu as pltpu
from jax.experimental.pallas import tpu_sc as plsc

B = 64
A = 64
BOARD = 64
N = B * A
DECAY = 0.8
WINDOW = 10.0
RADIUS = 3.0
FLOW = 0.5

GRAN = 16
NROWS = N * BOARD * BOARD // GRAN
NC, NS, L = 2, 16, 16
NW = NC * NS
PW = N // NW


@functools.cache
def _get_sc_safety():
    mesh = plsc.VectorSubcoreMesh(core_axis_name="c", subcore_axis_name="s")
    cp = pltpu.CompilerParams(
        needs_layout_passes=False, use_tc_tiling_on_sc=False)

    @functools.partial(
        pl.kernel,
        compiler_params=cp,
        out_type=jax.ShapeDtypeStruct((N,), jnp.float32),
        mesh=mesh,
        scratch_types=[
            pltpu.VMEM((PW,), jnp.float32),
            pltpu.VMEM((PW,), jnp.float32),
            pltpu.VMEM((PW,), jnp.float32),
            pltpu.VMEM((PW,), jnp.int32),
            pltpu.VMEM((PW,), jnp.int32),
            pltpu.VMEM((PW, GRAN), jnp.float32),
            pltpu.VMEM((PW,), jnp.float32),
            pltpu.SemaphoreType.DMA,
        ],
    )
    def _sc_safety(table_hbm, px_hbm, py_hbm, coll_hbm, out_hbm,
                   px_v, py_v, coll_v, row_v, col_v, rows_v, saf_v, sem):
        wid = lax.axis_index("s") * NC + lax.axis_index("c")
        base = wid * PW
        pltpu.sync_copy(px_hbm.at[pl.ds(base, PW)], px_v)
        pltpu.sync_copy(py_hbm.at[pl.ds(base, PW)], py_v)
        pltpu.sync_copy(coll_hbm.at[pl.ds(base, PW)], coll_v)

        for i in range(PW // L):
            xi = px_v[pl.ds(i * L, L)].astype(jnp.int32)
            yi = py_v[pl.ds(i * L, L)].astype(jnp.int32)
            xi = jnp.minimum(jnp.maximum(xi, 0), BOARD - 1)
            yi = jnp.minimum(jnp.maximum(yi, 0), BOARD - 1)
            lin = base + i * L + lax.iota(jnp.int32, L)
            row_v[pl.ds(i * L, L)] = (
                lin * (BOARD * BOARD // GRAN) + yi * (BOARD // GRAN)
                + (xi >> 4)
            )
            col_v[pl.ds(i * L, L)] = xi & (GRAN - 1)

        pltpu.async_copy(table_hbm.at[row_v], rows_v, sem).wait()

        for i in range(PW // L):
            ridx = i * L + lax.iota(jnp.int32, L)
            cidx = col_v[pl.ds(i * L, L)]
            g = plsc.load_gather(rows_v, [ridx, cidx])
            rate = g * (1.0 - 1.0 / WINDOW) + coll_v[pl.ds(i * L, L)] / WINDOW
            saf_v[pl.ds(i * L, L)] = 1.0 - jnp.minimum(
                jnp.maximum(rate, 0.0), 1.0)

        pltpu.sync_copy(saf_v, out_hbm.at[pl.ds(base, PW)])

    return _sc_safety


def _tc_bias_kernel(px_ref, py_ref, gx_ref, gy_ref, mx_ref, my_ref,
                    ppx_ref, ppy_ref, pxc_ref, pyc_ref, saf_ref, out_ref):
    px = px_ref[...]
    py = py_ref[...]
    gvx = gx_ref[...] - px
    gvy = gy_ref[...] - py
    gd = jnp.sqrt(gvx * gvx + gvy * gvy) + 1e-8
    nx = gvx / gd
    ny = gvy / gd

    dx = pxc_ref[...] - lax.broadcast_in_dim(px, (B, A, A), (0, 2))
    dy = pyc_ref[...] - lax.broadcast_in_dim(py, (B, A, A), (0, 2))
    dist = jnp.sqrt(dx * dx + dy * dy + 1e-12)
    mask = (dist <= RADIUS) & (dist > 0.1)
    w = jnp.where(mask, 1.0 / (dist + 1e-8), 0.0)
    wsum = jnp.sum(w, axis=-1) + 1e-8

    nmx = DECAY * mx_ref[...] + (1.0 - DECAY) * (px - ppx_ref[...])
    nmy = DECAY * my_ref[...] + (1.0 - DECAY) * (py - ppy_ref[...])
    cx = jnp.sum(w * lax.broadcast_in_dim(nmx, (B, A, A), (0, 2)),
                 axis=-1) / wsum
    cy = jnp.sum(w * lax.broadcast_in_dim(nmy, (B, A, A), (0, 2)),
                 axis=-1) / wsum

    saf = saf_ref[...]
    half_saf = 0.5 * saf
    out_ref[0] = half_saf
    out_ref[1] = (nx + 1.0) * half_saf + FLOW * cx
    out_ref[2] = (ny + 1.0) * half_saf + FLOW * cy
    out_ref[3] = (1.0 - nx) * half_saf - FLOW * cx
    out_ref[4] = (1.0 - ny) * half_saf - FLOW * cy


def kernel(positions, goals, prev_positions, agent_momentum,
           collision_history, collisions):
    px = positions[..., 0]
    py = positions[..., 1]
    gx = goals[..., 0]
    gy = goals[..., 1]
    ppx = prev_positions[..., 0]
    ppy = prev_positions[..., 1]
    mx = agent_momentum[..., 0]
    my = agent_momentum[..., 1]
    collf = collisions.astype(jnp.float32)
    table = collision_history.reshape(NROWS, GRAN)

    saf = _get_sc_safety()(
        table, px.reshape(N), py.reshape(N), collf.reshape(N)).reshape(B, A)

    out = pl.pallas_call(
        _tc_bias_kernel,
        out_shape=jax.ShapeDtypeStruct((5, B, A), jnp.float32),
    )(px, py, gx, gy, mx, my, ppx, ppy,
      px[:, :, None], py[:, :, None], saf)

    return jnp.transpose(out, (1, 2, 0))
